# Initial kernel scaffold; baseline (speedup 1.0000x reference)
import jax
import jax.numpy as jnp
from jax.experimental import pallas as pl


def kernel(z, idx, mu_prior, logvar_prior, logvar_uniform_prior, user_mu, user_logvar):
    raise NotImplementedError("write your pallas kernel here")


# speedup vs baseline: 1.4449x; 1.4449x over previous
import functools
import math

import jax
import jax.numpy as jnp
from jax import lax
from jax.experimental import pallas as pl
from jax.experimental.pallas import tpu as pltpu
from jax.experimental.pallas import tpu_sc as plsc

_NC, _NS = 2, 16
_CH = 128


def _sc_gather(user_mu, user_logvar, idx2):
    V, D = user_mu.shape
    nw = _NC * _NS
    n_ch = idx2.shape[0] // nw
    b_per_w = n_ch * _CH
    B = nw * b_per_w
    mesh = plsc.VectorSubcoreMesh(core_axis_name="c", subcore_axis_name="s")

    @functools.partial(
        pl.kernel,
        mesh=mesh,
        out_type=[
            jax.ShapeDtypeStruct((B, D), jnp.float32),
            jax.ShapeDtypeStruct((B, D), jnp.float32),
        ],
        scratch_types=[
            pltpu.VMEM((n_ch, _CH), jnp.int32),
            pltpu.VMEM((b_per_w, D), jnp.float32),
            pltpu.SemaphoreType.DMA,
        ],
    )
    def k(mu_hbm, lv_hbm, idx_hbm, mu_out, lv_out, idx_v, rows_v, sem):
        wid = lax.axis_index("s") * _NC + lax.axis_index("c")
        base = wid * b_per_w
        pltpu.sync_copy(idx_hbm.at[pl.ds(wid * n_ch, n_ch)], idx_v)
        for tbl, out in ((mu_hbm, mu_out), (lv_hbm, lv_out)):
            cps = [
                pltpu.async_copy(
                    tbl.at[idx_v.at[j]], rows_v.at[pl.ds(j * _CH, _CH)], sem
                )
                for j in range(n_ch)
            ]
            for cp in cps:
                cp.wait()
            pltpu.sync_copy(rows_v, out.at[pl.ds(base, b_per_w)])

    return k(user_mu, user_logvar, idx2)


def _tc_math(z, mu_e, lv_e, mu_p, lv_p, lv_u):
    B, D = z.shape
    blk = 2048
    c1 = math.log(1.0 / 5.0 - 1.0 / 20.0) - 0.5 * math.log(2.0 * math.pi)
    c2 = math.log(4.0 / 5.0 - 1.0 / 20.0) - 0.5 * math.log(2.0 * math.pi)
    c3 = math.log(1.0 / 10.0) - 0.5 * math.log(2.0 * math.pi)

    def body(z_ref, mu_ref, lv_ref, mup_ref, lvp_ref, lvu_ref, o_ref):
        zz = z_ref[...]
        mup = mup_ref[...]
        lvp = lvp_ref[...]
        lvu = lvu_ref[...]
        zp2 = (zz - mup) ** 2
        d1 = -0.5 * lvp - 0.5 * zp2 * jnp.exp(-lvp) + c1
        d3 = -0.5 * lvu - 0.5 * zp2 * jnp.exp(-lvu) + c3
        ze = zz - mu_ref[...]
        lve = lv_ref[...]
        d2 = -0.5 * lve - 0.5 * ze * ze * jnp.exp(-lve) + c2
        m = jnp.maximum(d1, jnp.maximum(d2, d3))
        o_ref[...] = m + jnp.log(
            jnp.exp(d1 - m) + jnp.exp(d2 - m) + jnp.exp(d3 - m)
        )

    bs = pl.BlockSpec((blk, D), lambda i: (i, 0))
    ps = pl.BlockSpec((1, D), lambda i: (0, 0))
    return pl.pallas_call(
        body,
        grid=(B // blk,),
        in_specs=[bs, bs, bs, ps, ps, ps],
        out_specs=bs,
        out_shape=jax.ShapeDtypeStruct((B, D), jnp.float32),
    )(z, mu_e, lv_e, mu_p, lv_p, lv_u)


def kernel(z, idx, mu_prior, logvar_prior, logvar_uniform_prior, user_mu, user_logvar):
    B, D = z.shape
    idx2 = idx.astype(jnp.int32).reshape(-1, _CH)
    mu_e, lv_e = _sc_gather(user_mu, user_logvar, idx2)
    return _tc_math(
        z,
        mu_e,
        lv_e,
        mu_prior.reshape(1, D),
        logvar_prior.reshape(1, D),
        logvar_uniform_prior.reshape(1, D),
    )
